# baseline (speedup 1.0000x reference)
import functools

import jax
import jax.numpy as jnp
from jax import lax
from jax.experimental import pallas as pl
from jax.experimental.pallas import tpu as pltpu
from jax.experimental.pallas import tpu_sc as plsc

NC = 2
NS = 16
NW = NC * NS
LANES = 16
CH = 128
NBUF = 2


@functools.partial(jax.jit, static_argnames=("f", "bt", "d"))
def _embed_sum(xT, W0r, W1r, f, bt, d):
    n_blocks = f * bt
    per_w = n_blocks // NW
    mesh = plsc.VectorSubcoreMesh(
        core_axis_name="c", subcore_axis_name="s",
        num_cores=NC, num_subcores=NS)

    @functools.partial(
        pl.kernel,
        mesh=mesh,
        compiler_params=pltpu.CompilerParams(needs_layout_passes=False),
        out_type=jax.ShapeDtypeStruct((f, d, bt * CH), jnp.float32),
        scratch_types=[
            pltpu.VMEM((NBUF, CH), jnp.int32),
            pltpu.VMEM((NBUF, CH), jnp.int32),
            pltpu.VMEM((NBUF, CH, 128), jnp.float32),
            pltpu.VMEM((NBUF, CH, 128), jnp.float32),
            pltpu.VMEM((NBUF, d, CH), jnp.float32),
            pltpu.SemaphoreType.DMA((NBUF,)),
            pltpu.SemaphoreType.DMA((NBUF,)),
            pltpu.SemaphoreType.DMA((NBUF,)),
        ],
    )
    def body(x_hbm, w0_hbm, w1_hbm, out_hbm, idxq, qoff, r0, r1, o, semi,
             semg, semo):
        wid = lax.axis_index("s") * NC + lax.axis_index("c")
        blk0 = wid * per_w

        def stage_idx(k, b):
            blk = blk0 + k
            pltpu.async_copy(
                x_hbm.at[blk // bt, pl.ds((blk % bt) * CH, CH)],
                idxq.at[b], semi.at[b])

        def fire_gathers(b):
            pltpu.make_async_copy(
                x_hbm.at[0, pl.ds(0, CH)], idxq.at[b], semi.at[b]).wait()

            def split(g, carry):
                v = idxq[b, pl.ds(g * LANES, LANES)]
                qoff[b, pl.ds(g * LANES, LANES)] = (v & 3) * 32
                idxq[b, pl.ds(g * LANES, LANES)] = v >> 2
                return carry

            lax.fori_loop(0, CH // LANES, split, 0)
            pltpu.async_copy(w0_hbm.at[idxq.at[b]], r0.at[b], semg.at[b])
            pltpu.async_copy(w1_hbm.at[idxq.at[b]], r1.at[b], semg.at[b])

        for b in range(NBUF):
            stage_idx(b, b)
        fire_gathers(0)

        def block_body(k, carry):
            b = k % NBUF
            blk = blk0 + k
            pltpu.make_async_copy(
                w0_hbm.at[idxq.at[b]], r0.at[b], semg.at[b]).wait()
            pltpu.make_async_copy(
                w0_hbm.at[idxq.at[b]], r1.at[b], semg.at[b]).wait()

            @pl.when(k + 1 < per_w)
            def _():
                fire_gathers((k + 1) % NBUF)

            @pl.when(k >= NBUF)
            def _():
                pltpu.make_async_copy(
                    o.at[b], out_hbm.at[0, pl.ds(0, d), pl.ds(0, CH)],
                    semo.at[b]).wait()

            def group_body(g, carry2):
                rows = lax.iota(jnp.int32, LANES) + g * LANES
                cols = qoff[b, pl.ds(g * LANES, LANES)]
                for dd in range(32):
                    v0 = plsc.load_gather(r0.at[b], [rows, cols + dd])
                    v1 = plsc.load_gather(r1.at[b], [rows, cols + dd])
                    o[b, dd, pl.ds(g * LANES, LANES)] = v0 + v1
                return carry2

            lax.fori_loop(0, CH // LANES, group_body, 0)

            pltpu.async_copy(
                o.at[b],
                out_hbm.at[blk // bt, pl.ds(0, d), pl.ds((blk % bt) * CH, CH)],
                semo.at[b])

            @pl.when(k + NBUF < per_w)
            def _():
                stage_idx(k + NBUF, b)
            return carry

        lax.fori_loop(0, per_w, block_body, 0)

        for b in range(NBUF):
            pltpu.make_async_copy(
                o.at[b], out_hbm.at[0, pl.ds(0, d), pl.ds(0, CH)],
                semo.at[b]).wait()

    return body(xT, W0r, W1r)


def kernel(x, W0, W1):
    bsz, f = x.shape
    v, d = W0.shape
    bt = bsz // CH
    xT = x.T
    w0r = W0.reshape(v // 4, 128)
    w1r = W1.reshape(v // 4, 128)
    out_t = _embed_sum(xT, w0r, w1r, f, bt, d)
    return out_t.transpose(2, 0, 1)
